# baseline (speedup 1.0000x reference)
import functools

import jax
import jax.numpy as jnp
from jax import lax
from jax.experimental import pallas as pl
from jax.experimental.pallas import tpu as pltpu
from jax.experimental.pallas import tpu_sc as plsc

B = 16384
L = 20
D = 64
LANES = 16

NC = 2
NS = 16
NW = NC * NS

BPW = B // NW
CH = 32
NCHUNK = BPW // CH
ROWS = CH * L
GSZ = 128
NG = ROWS // GSZ


def _body(idx_hbm, table_hbm, out_hbm, idx_v, rows_v, out_v, sem):
    wid = lax.axis_index("s") * NC + lax.axis_index("c")
    base = wid * BPW

    pltpu.sync_copy(idx_hbm.at[pl.ds(base * L, BPW * L)], idx_v)

    for g in range(NCHUNK):
        descs = [
            pltpu.async_copy(
                table_hbm.at[idx_v.at[pl.ds(g * ROWS + j * GSZ, GSZ)]],
                rows_v.at[pl.ds(j * GSZ, GSZ)],
                sem,
            )
            for j in range(NG)
        ]
        for d in descs:
            d.wait()

        def reduce_row(c, carry):
            for dd in range(D // LANES):
                acc = jnp.zeros((LANES,), jnp.float32)
                for t in range(L):
                    acc = acc + rows_v[c * L + t, pl.ds(dd * LANES, LANES)]
                out_v[c, pl.ds(dd * LANES, LANES)] = acc * (1.0 / L)
            return carry

        lax.fori_loop(0, CH, reduce_row, 0)
        pltpu.sync_copy(out_v, out_hbm.at[pl.ds(base + g * CH, CH)])


@jax.jit
def _run(idx_flat, table):
    mesh = plsc.VectorSubcoreMesh(core_axis_name="c", subcore_axis_name="s")
    f = pl.kernel(
        _body,
        out_type=jax.ShapeDtypeStruct((B, D), jnp.float32),
        mesh=mesh,
        scratch_types=[
            pltpu.VMEM((BPW * L,), jnp.int32),
            pltpu.VMEM((ROWS, D), jnp.float32),
            pltpu.VMEM((CH, D), jnp.float32),
            pltpu.SemaphoreType.DMA,
        ],
        compiler_params=pltpu.CompilerParams(use_tc_tiling_on_sc=False),
    )
    return f(idx_flat, table)


def kernel(indices, table):
    idx_flat = indices.astype(jnp.int32).reshape(B * L)
    return _run(idx_flat, table.astype(jnp.float32))


# speedup vs baseline: 1.1007x; 1.1007x over previous
import functools

import jax
import jax.numpy as jnp
from jax import lax
from jax.experimental import pallas as pl
from jax.experimental.pallas import tpu as pltpu
from jax.experimental.pallas import tpu_sc as plsc

B = 16384
L = 20
D = 64
LANES = 16

NC = 2
NS = 16
NW = NC * NS

BPW = B // NW
CH = 32
NCHUNK = BPW // CH
ROWS = CH * L
GSZ = 128
NG = ROWS // GSZ


def _body(idx_hbm, table_hbm, out_hbm, idx_v, rows_v, out_v, sem0, sem1):
    wid = lax.axis_index("s") * NC + lax.axis_index("c")
    base = wid * BPW
    sems = (sem0, sem1)

    pltpu.sync_copy(idx_hbm.at[pl.ds(base * L, BPW * L)], idx_v)

    def fire(g, slot):
        for j in range(NG):
            pltpu.async_copy(
                table_hbm.at[idx_v.at[pl.ds(g * ROWS + j * GSZ, GSZ)]],
                rows_v.at[slot, pl.ds(j * GSZ, GSZ)],
                sems[slot],
            )

    def drain(slot):
        pltpu.make_async_copy(
            table_hbm.at[pl.ds(0, ROWS)], rows_v.at[slot], sems[slot]
        ).wait()

    def reduce_store(g, slot):
        @plsc.parallel_loop(0, CH, 1, unroll=2)
        def _red(c):
            for dd in range(D // LANES):
                acc = jnp.zeros((LANES,), jnp.float32)
                for t in range(L):
                    acc = acc + rows_v[slot, c * L + t, pl.ds(dd * LANES, LANES)]
                out_v[slot, c, pl.ds(dd * LANES, LANES)] = acc * (1.0 / L)

        pltpu.sync_copy(out_v.at[slot], out_hbm.at[pl.ds(base + g * CH, CH)])

    fire(0, 0)

    def step(k, carry):
        g = 2 * k
        fire(g + 1, 1)
        drain(0)
        reduce_store(g, 0)

        @pl.when(k < NCHUNK // 2 - 1)
        def _():
            fire(g + 2, 0)

        drain(1)
        reduce_store(g + 1, 1)
        return carry

    lax.fori_loop(0, NCHUNK // 2, step, 0)


@jax.jit
def _run(idx_flat, table):
    mesh = plsc.VectorSubcoreMesh(core_axis_name="c", subcore_axis_name="s")
    f = pl.kernel(
        _body,
        out_type=jax.ShapeDtypeStruct((B, D), jnp.float32),
        mesh=mesh,
        scratch_types=[
            pltpu.VMEM((BPW * L,), jnp.int32),
            pltpu.VMEM((2, ROWS, D), jnp.float32),
            pltpu.VMEM((2, CH, D), jnp.float32),
            pltpu.SemaphoreType.DMA,
            pltpu.SemaphoreType.DMA,
        ],
        compiler_params=pltpu.CompilerParams(use_tc_tiling_on_sc=False),
    )
    return f(idx_flat, table)


def kernel(indices, table):
    idx_flat = indices.astype(jnp.int32).reshape(B * L)
    return _run(idx_flat, table.astype(jnp.float32))
